# baseline (speedup 1.0000x reference)
import functools

import jax
import jax.numpy as jnp
from jax import lax
from jax.experimental import pallas as pl
from jax.experimental.pallas import tpu as pltpu
from jax.experimental.pallas import tpu_sc as plsc

N = 10000
E = 320000
D_IN = 128
H = 16
L = 4
G = 64

NW = 32
CHUNK = 128
NB = 8
CH = 80
E_PAD = NW * CH * CHUNK
N_PAD = 10112
RPT = N_PAD // 16
NR = N_PAD // 8
N_DUMMY = N

_f32 = jnp.float32


def _sc_agg(hm_lin, src3, dst3, zeros_tile):
    mesh = plsc.VectorSubcoreMesh(core_axis_name="c", subcore_axis_name="s")

    @functools.partial(
        pl.kernel,
        mesh=mesh,
        compiler_params=pltpu.CompilerParams(use_tc_tiling_on_sc=False),
        out_type=jax.ShapeDtypeStruct((2 * N_PAD, H), _f32),
        scratch_types=[
            pltpu.VMEM((NB, CHUNK), jnp.int32),
            pltpu.VMEM((NB, CHUNK), jnp.int32),
            pltpu.VMEM((NB, CHUNK, H), _f32),
            pltpu.VMEM((RPT, H), _f32),
            pltpu.VMEM_SHARED((N_PAD, H), _f32),
            pltpu.VMEM_SHARED((N_PAD, H), _f32),
            [pltpu.SemaphoreType.DMA] * NB,
        ],
    )
    def k(hm_hbm, src_hbm, dst_hbm, zeros_hbm, out_hbm,
          src_v, dst_v, rows_v, stage_v, hm_sh, agg_sh, sems):
        c = lax.axis_index("c")
        s = lax.axis_index("s")
        w = c * 16 + s
        r0 = s * RPT

        pltpu.sync_copy(zeros_hbm, stage_v)
        pltpu.sync_copy(stage_v, agg_sh.at[pl.ds(r0, RPT)])
        pltpu.sync_copy(hm_hbm.at[pl.ds(r0, RPT)], stage_v)
        pltpu.sync_copy(stage_v, hm_sh.at[pl.ds(r0, RPT)])
        plsc.subcore_barrier()

        def body(b, carry):
            pltpu.sync_copy(src_hbm.at[pl.ds(w * CH + b * NB, NB)], src_v)
            pltpu.sync_copy(dst_hbm.at[pl.ds(w * CH + b * NB, NB)], dst_v)
            gathers = [
                pltpu.async_copy(hm_sh.at[src_v.at[g]], rows_v.at[g], sems[g])
                for g in range(NB)
            ]
            scatters = []
            for g in range(NB):
                gathers[g].wait()
                scatters.append(
                    pltpu.async_copy(rows_v.at[g], agg_sh.at[dst_v.at[g]],
                                     sems[g], add=True))
            for sc in scatters:
                sc.wait()
            return carry

        lax.fori_loop(0, CH // NB, body, 0)
        plsc.subcore_barrier()
        pltpu.sync_copy(agg_sh.at[pl.ds(r0, RPT)],
                        out_hbm.at[pl.ds(c * N_PAD + r0, RPT)])

    return k(hm_lin, src3, dst3, zeros_tile)


def _dot(a, b):
    return jnp.dot(a, b, preferred_element_type=_f32)


def _dot_t(a, b):
    return lax.dot_general(a, b, (((0,), (0,)), ((), ())),
                           preferred_element_type=_f32)


def _pack(h):
    return jnp.concatenate([h[NR * j:NR * (j + 1), :] for j in range(8)],
                           axis=1)


def _unpack(hp):
    return jnp.concatenate([hp[:, H * j:H * (j + 1)] for j in range(8)],
                           axis=0)


def _embed_body(x_ref, we_ref, bdm_ref, hp_ref, hmp_ref):
    h = _dot(x_ref[...], we_ref[...])
    hf = jnp.concatenate(
        [h, jnp.zeros((N_PAD - N, H), _f32)], axis=0)
    hp = _pack(hf)
    hp_ref[...] = hp
    hmp_ref[...] = _dot(hp, bdm_ref[...])


def _embed_call(x, W_embed, bdm0):
    return pl.pallas_call(
        _embed_body,
        out_shape=(jax.ShapeDtypeStruct((NR, 128), _f32),
                   jax.ShapeDtypeStruct((NR, 128), _f32)),
    )(x, W_embed, bdm0)


def _gru_packed(hp, aggp, bdwz, bduz, bzp, bdwr, bdur, brp, bdwh, bduh, bhp):
    z = jax.nn.sigmoid(_dot(aggp, bdwz) + _dot(hp, bduz) + bzp)
    r = jax.nn.sigmoid(_dot(aggp, bdwr) + _dot(hp, bdur) + brp)
    h_tilde = jnp.tanh(_dot(aggp, bdwh) + _dot(r * hp, bduh) + bhp)
    return (1.0 - z) * hp + z * h_tilde


def _gru_body(h_ref, p_ref, wz_ref, uz_ref, bz_ref, wr_ref, ur_ref, br_ref,
              wh_ref, uh_ref, bh_ref, bdm_ref, hout_ref, hm_ref):
    p = p_ref[...]
    aggp = p[:NR] + p[NR:]
    hp = _gru_packed(h_ref[...], aggp, wz_ref[...], uz_ref[...], bz_ref[...],
                     wr_ref[...], ur_ref[...], br_ref[...],
                     wh_ref[...], uh_ref[...], bh_ref[...])
    hout_ref[...] = hp
    hm_ref[...] = _dot(hp, bdm_ref[...])


def _gru_call(h, parts, *ws):
    return pl.pallas_call(
        _gru_body,
        out_shape=(jax.ShapeDtypeStruct((NR, 128), _f32),
                   jax.ShapeDtypeStruct((NR, 128), _f32)),
    )(h, parts, *ws)


def _exchange(hp, mapb_ref, bdexa, bdexb, bexp):
    ids_g = lax.broadcasted_iota(jnp.int32, (G, NR), 0)
    sums = jnp.zeros((G, H), _f32)
    cnt = jnp.zeros((G, 1), _f32)
    ohs = []
    for j in range(8):
        mj = mapb_ref[pl.ds(j, 1), :]
        ohj = (ids_g == mj).astype(_f32)
        ohs.append(ohj)
        sums = sums + _dot(ohj, hp[:, H * j:H * (j + 1)])
        cnt = cnt + jnp.sum(ohj, axis=1, keepdims=True)
    mean = sums / jnp.maximum(cnt, 1.0)
    pn = jnp.concatenate([_dot_t(ohj, mean) for ohj in ohs], axis=1)
    return hp + jnp.tanh(_dot(hp, bdexa) + _dot(pn, bdexb) + bexp)


def _gru_ex_body(h_ref, p_ref, wz_ref, uz_ref, bz_ref, wr_ref, ur_ref,
                 br_ref, wh_ref, uh_ref, bh_ref, mapb_ref, bdexa_ref,
                 bdexb_ref, bex_ref, bdm_ref, hout_ref, hm_ref):
    p = p_ref[...]
    aggp = p[:NR] + p[NR:]
    hp = _gru_packed(h_ref[...], aggp, wz_ref[...], uz_ref[...], bz_ref[...],
                     wr_ref[...], ur_ref[...], br_ref[...],
                     wh_ref[...], uh_ref[...], bh_ref[...])
    hp = _exchange(hp, mapb_ref, bdexa_ref[...], bdexb_ref[...], bex_ref[...])
    hout_ref[...] = hp
    hm_ref[...] = _dot(hp, bdm_ref[...])


def _gru_ex_call(h, parts, *ws):
    return pl.pallas_call(
        _gru_ex_body,
        out_shape=(jax.ShapeDtypeStruct((NR, 128), _f32),
                   jax.ShapeDtypeStruct((NR, 128), _f32)),
    )(h, parts, *ws)


def _gru_ex_last_body(h_ref, p_ref, wz_ref, uz_ref, bz_ref, wr_ref, ur_ref,
                      br_ref, wh_ref, uh_ref, bh_ref, mapb_ref, bdexa_ref,
                      bdexb_ref, bex_ref, hout_ref):
    p = p_ref[...]
    aggp = p[:NR] + p[NR:]
    hp = _gru_packed(h_ref[...], aggp, wz_ref[...], uz_ref[...], bz_ref[...],
                     wr_ref[...], ur_ref[...], br_ref[...],
                     wh_ref[...], uh_ref[...], bh_ref[...])
    hp = _exchange(hp, mapb_ref, bdexa_ref[...], bdexb_ref[...], bex_ref[...])
    hout_ref[...] = _unpack(hp)


def _gru_ex_last_call(h, parts, *ws):
    return pl.pallas_call(
        _gru_ex_last_body,
        out_shape=jax.ShapeDtypeStruct((N_PAD, H), _f32),
    )(h, parts, *ws)


def kernel(x, edge_index, node_to_graph_map, W_embed, W_msg,
           Wz, Uz, bz, Wr, Ur, br, Wh, Uh, bh, W_ex, b_ex):
    wstack = jnp.concatenate(
        [W_msg, Wz, Uz, Wr, Ur, Wh, Uh,
         W_ex[:, :H, :], W_ex[:, H:, :]], axis=0)
    bd = jnp.einsum('ab,lcd->lacbd', jnp.eye(8, dtype=_f32),
                    wstack).reshape(-1, 128, 128)
    bdm = bd[0:4]
    bdwz, bduz = bd[4:8], bd[8:12]
    bdwr, bdur = bd[12:16], bd[16:20]
    bdwh, bduh = bd[20:24], bd[24:28]
    bdexa, bdexb = bd[28:30], bd[30:32]
    bstack = jnp.concatenate([bz, br, bh, b_ex], axis=0)
    bp = jnp.concatenate([bstack] * 8, axis=1)
    bzp, brp, bhp = bp[0:4], bp[4:8], bp[8:12]
    bexp = bp[12:14]

    src = edge_index[0]
    dst = jnp.pad(edge_index[1], (0, E_PAD - E), constant_values=N_DUMMY)
    src = jnp.pad(src, (0, E_PAD - E))
    srcq = ((src % NR) * 8 + src // NR).reshape(NW * CH, CHUNK)
    dstq = ((dst % NR) * 8 + dst // NR).reshape(NW * CH, CHUNK)
    mapb = jnp.pad(node_to_graph_map, (0, N_PAD - N),
                   constant_values=G).reshape(8, NR)
    zeros_tile = jnp.zeros((RPT, H), _f32)

    hp, hmp = _embed_call(x, W_embed, bdm[0])
    for l in range(L):
        parts = _sc_agg(jnp.reshape(hmp, (N_PAD, H)), srcq, dstq, zeros_tile)
        partsp = jnp.reshape(parts, (2 * NR, 128))
        gw = (bdwz[l], bduz[l], bzp[l:l + 1], bdwr[l], bdur[l], brp[l:l + 1],
              bdwh[l], bduh[l], bhp[l:l + 1])
        if l == L - 1:
            h_final = _gru_ex_last_call(hp, partsp, *gw, mapb, bdexa[1],
                                        bdexb[1], bexp[1:2])
        elif l % 2 == 1:
            hp, hmp = _gru_ex_call(hp, partsp, *gw, mapb, bdexa[0],
                                   bdexb[0], bexp[0:1], bdm[l + 1])
        else:
            hp, hmp = _gru_call(hp, partsp, *gw, bdm[l + 1])
    return h_final[:N]


# speedup vs baseline: 2.2307x; 2.2307x over previous
import functools

import jax
import jax.numpy as jnp
from jax import lax
from jax.experimental import pallas as pl
from jax.experimental.pallas import tpu as pltpu
from jax.experimental.pallas import tpu_sc as plsc

N = 10000
E = 320000
D_IN = 128
H = 16
L = 4
G = 64

NW = 32
CHUNK = 128
NB = 8
CH = 80
E_PAD = NW * CH * CHUNK
N_PAD = 10112
RPT = N_PAD // 16
NR = N_PAD // 8
N_DUMMY = N

_f32 = jnp.float32


def _sc_agg(hm_lin, src3, dst3, zeros_tile):
    mesh = plsc.VectorSubcoreMesh(core_axis_name="c", subcore_axis_name="s")

    @functools.partial(
        pl.kernel,
        mesh=mesh,
        compiler_params=pltpu.CompilerParams(use_tc_tiling_on_sc=False),
        out_type=jax.ShapeDtypeStruct((2 * N_PAD, H), _f32),
        scratch_types=[
            pltpu.VMEM((NB, CHUNK), jnp.int32),
            pltpu.VMEM((NB, CHUNK), jnp.int32),
            pltpu.VMEM((NB, CHUNK, H), _f32),
            pltpu.VMEM((RPT, H), _f32),
            pltpu.VMEM_SHARED((N_PAD, H), _f32),
            pltpu.VMEM_SHARED((N_PAD, H), _f32),
            [pltpu.SemaphoreType.DMA] * NB,
        ],
    )
    def k(hm_hbm, src_hbm, dst_hbm, zeros_hbm, out_hbm,
          src_v, dst_v, rows_v, stage_v, hm_sh, agg_sh, sems):
        c = lax.axis_index("c")
        s = lax.axis_index("s")
        w = c * 16 + s
        r0 = s * RPT

        pltpu.sync_copy(zeros_hbm, stage_v)
        pltpu.sync_copy(stage_v, agg_sh.at[pl.ds(r0, RPT)])
        pltpu.sync_copy(hm_hbm.at[pl.ds(r0, RPT)], stage_v)
        pltpu.sync_copy(stage_v, hm_sh.at[pl.ds(r0, RPT)])
        plsc.subcore_barrier()

        def body(b, carry):
            pltpu.sync_copy(src_hbm.at[pl.ds(w * CH + b * NB, NB)], src_v)
            pltpu.sync_copy(dst_hbm.at[pl.ds(w * CH + b * NB, NB)], dst_v)
            gathers = [
                pltpu.async_copy(hm_sh.at[src_v.at[g]], rows_v.at[g], sems[g])
                for g in range(NB)
            ]
            scatters = []
            for g in range(NB):
                gathers[g].wait()
                scatters.append(
                    pltpu.async_copy(rows_v.at[g], agg_sh.at[dst_v.at[g]],
                                     sems[g], add=True))
            for sc in scatters:
                sc.wait()
            return carry

        lax.fori_loop(0, CH // NB, body, 0)
        plsc.subcore_barrier()
        pltpu.sync_copy(agg_sh.at[pl.ds(r0, RPT)],
                        out_hbm.at[pl.ds(c * N_PAD + r0, RPT)])

    return k(hm_lin, src3, dst3, zeros_tile)


def _dot(a, b):
    return jnp.dot(a, b, preferred_element_type=_f32)


def _bd(w_ref):
    w16 = w_ref[...]
    rows = jnp.concatenate([w16] * 8, axis=0)
    full = jnp.concatenate([rows] * 8, axis=1)
    ri = lax.broadcasted_iota(jnp.int32, (128, 128), 0) // H
    ci = lax.broadcasted_iota(jnp.int32, (128, 128), 1) // H
    return full * (ri == ci).astype(_f32)


def _bias(b_ref):
    return jnp.concatenate([b_ref[...]] * 8, axis=1)


def _dot_t(a, b):
    return lax.dot_general(a, b, (((0,), (0,)), ((), ())),
                           preferred_element_type=_f32)


def _pack(h):
    return jnp.concatenate([h[NR * j:NR * (j + 1), :] for j in range(8)],
                           axis=1)


def _unpack(hp):
    return jnp.concatenate([hp[:, H * j:H * (j + 1)] for j in range(8)],
                           axis=0)


def _embed_body(x_ref, we_ref, wm_ref, hp_ref, hmp_ref):
    h = _dot(x_ref[...], we_ref[...])
    hf = jnp.concatenate(
        [h, jnp.zeros((N_PAD - N, H), _f32)], axis=0)
    hp = _pack(hf)
    hp_ref[...] = hp
    hmp_ref[...] = _dot(hp, _bd(wm_ref))


def _embed_call(x, W_embed, wm0):
    return pl.pallas_call(
        _embed_body,
        out_shape=(jax.ShapeDtypeStruct((NR, 128), _f32),
                   jax.ShapeDtypeStruct((NR, 128), _f32)),
    )(x, W_embed, wm0)


def _gru_packed(hp, aggp, wz_ref, uz_ref, bz_ref, wr_ref, ur_ref, br_ref,
                wh_ref, uh_ref, bh_ref):
    z = jax.nn.sigmoid(_dot(aggp, _bd(wz_ref)) + _dot(hp, _bd(uz_ref))
                       + _bias(bz_ref))
    r = jax.nn.sigmoid(_dot(aggp, _bd(wr_ref)) + _dot(hp, _bd(ur_ref))
                       + _bias(br_ref))
    h_tilde = jnp.tanh(_dot(aggp, _bd(wh_ref)) + _dot(r * hp, _bd(uh_ref))
                       + _bias(bh_ref))
    return (1.0 - z) * hp + z * h_tilde


def _gru_body(h_ref, p_ref, wz_ref, uz_ref, bz_ref, wr_ref, ur_ref, br_ref,
              wh_ref, uh_ref, bh_ref, wm_ref, hout_ref, hm_ref):
    p = p_ref[...]
    aggp = p[:NR] + p[NR:]
    hp = _gru_packed(h_ref[...], aggp, wz_ref, uz_ref, bz_ref,
                     wr_ref, ur_ref, br_ref, wh_ref, uh_ref, bh_ref)
    hout_ref[...] = hp
    hm_ref[...] = _dot(hp, _bd(wm_ref))


def _gru_call(h, parts, *ws):
    return pl.pallas_call(
        _gru_body,
        out_shape=(jax.ShapeDtypeStruct((NR, 128), _f32),
                   jax.ShapeDtypeStruct((NR, 128), _f32)),
    )(h, parts, *ws)


def _exchange(hp, mapb_ref, wexa_ref, wexb_ref, bex_ref):
    ids_g = lax.broadcasted_iota(jnp.int32, (G, NR), 0)
    sums = jnp.zeros((G, H), _f32)
    cnt = jnp.zeros((G, 1), _f32)
    ohs = []
    for j in range(8):
        mj = mapb_ref[pl.ds(j, 1), :]
        ohj = (ids_g == mj).astype(_f32)
        ohs.append(ohj)
        sums = sums + _dot(ohj, hp[:, H * j:H * (j + 1)])
        cnt = cnt + jnp.sum(ohj, axis=1, keepdims=True)
    mean = sums / jnp.maximum(cnt, 1.0)
    pn = jnp.concatenate([_dot_t(ohj, mean) for ohj in ohs], axis=1)
    return hp + jnp.tanh(_dot(hp, _bd(wexa_ref)) + _dot(pn, _bd(wexb_ref))
                         + _bias(bex_ref))


def _gru_ex_body(h_ref, p_ref, wz_ref, uz_ref, bz_ref, wr_ref, ur_ref,
                 br_ref, wh_ref, uh_ref, bh_ref, mapb_ref, wexa_ref,
                 wexb_ref, bex_ref, wm_ref, hout_ref, hm_ref):
    p = p_ref[...]
    aggp = p[:NR] + p[NR:]
    hp = _gru_packed(h_ref[...], aggp, wz_ref, uz_ref, bz_ref,
                     wr_ref, ur_ref, br_ref, wh_ref, uh_ref, bh_ref)
    hp = _exchange(hp, mapb_ref, wexa_ref, wexb_ref, bex_ref)
    hout_ref[...] = hp
    hm_ref[...] = _dot(hp, _bd(wm_ref))


def _gru_ex_call(h, parts, *ws):
    return pl.pallas_call(
        _gru_ex_body,
        out_shape=(jax.ShapeDtypeStruct((NR, 128), _f32),
                   jax.ShapeDtypeStruct((NR, 128), _f32)),
    )(h, parts, *ws)


def _gru_ex_last_body(h_ref, p_ref, wz_ref, uz_ref, bz_ref, wr_ref, ur_ref,
                      br_ref, wh_ref, uh_ref, bh_ref, mapb_ref, wexa_ref,
                      wexb_ref, bex_ref, hout_ref):
    p = p_ref[...]
    aggp = p[:NR] + p[NR:]
    hp = _gru_packed(h_ref[...], aggp, wz_ref, uz_ref, bz_ref,
                     wr_ref, ur_ref, br_ref, wh_ref, uh_ref, bh_ref)
    hp = _exchange(hp, mapb_ref, wexa_ref, wexb_ref, bex_ref)
    hout_ref[...] = _unpack(hp)


def _gru_ex_last_call(h, parts, *ws):
    return pl.pallas_call(
        _gru_ex_last_body,
        out_shape=jax.ShapeDtypeStruct((N_PAD, H), _f32),
    )(h, parts, *ws)


def kernel(x, edge_index, node_to_graph_map, W_embed, W_msg,
           Wz, Uz, bz, Wr, Ur, br, Wh, Uh, bh, W_ex, b_ex):
    src = edge_index[0]
    dst = jnp.pad(edge_index[1], (0, E_PAD - E), constant_values=N_DUMMY)
    src = jnp.pad(src, (0, E_PAD - E))
    srcq = ((src % NR) * 8 + src // NR).reshape(NW * CH, CHUNK)
    dstq = ((dst % NR) * 8 + dst // NR).reshape(NW * CH, CHUNK)
    mapb = jnp.pad(node_to_graph_map, (0, N_PAD - N),
                   constant_values=G).reshape(8, NR)
    zeros_tile = jnp.zeros((RPT, H), _f32)

    hp, hmp = _embed_call(x, W_embed, W_msg[0])
    for l in range(L):
        parts = _sc_agg(jnp.reshape(hmp, (N_PAD, H)), srcq, dstq, zeros_tile)
        partsp = jnp.reshape(parts, (2 * NR, 128))
        gw = (Wz[l], Uz[l], bz[l].reshape(1, H), Wr[l], Ur[l],
              br[l].reshape(1, H), Wh[l], Uh[l], bh[l].reshape(1, H))
        ex_i = l // 2
        if l == L - 1:
            h_final = _gru_ex_last_call(hp, partsp, *gw, mapb,
                                        W_ex[ex_i, :H], W_ex[ex_i, H:],
                                        b_ex[ex_i].reshape(1, H))
        elif l % 2 == 1:
            hp, hmp = _gru_ex_call(hp, partsp, *gw, mapb,
                                   W_ex[ex_i, :H], W_ex[ex_i, H:],
                                   b_ex[ex_i].reshape(1, H), W_msg[l + 1])
        else:
            hp, hmp = _gru_call(hp, partsp, *gw, W_msg[l + 1])
    return h_final[:N]


# speedup vs baseline: 2.4787x; 1.1112x over previous
import functools

import jax
import jax.numpy as jnp
from jax import lax
from jax.experimental import pallas as pl
from jax.experimental.pallas import tpu as pltpu
from jax.experimental.pallas import tpu_sc as plsc

N = 10000
E = 320000
D_IN = 128
H = 16
L = 4
G = 64

NW = 32
CHUNK = 128
NB = 8
CH = 80
E_PAD = NW * CH * CHUNK
N_PAD = 10112
RPT = N_PAD // 16
NR = N_PAD // 8
N_DUMMY = N

_f32 = jnp.float32


def _sc_agg(hm_lin, src3, dst3, zeros_tile):
    mesh = plsc.VectorSubcoreMesh(core_axis_name="c", subcore_axis_name="s")

    @functools.partial(
        pl.kernel,
        mesh=mesh,
        compiler_params=pltpu.CompilerParams(use_tc_tiling_on_sc=False),
        out_type=jax.ShapeDtypeStruct((2 * N_PAD, H), _f32),
        scratch_types=[
            pltpu.VMEM((2, NB, CHUNK), jnp.int32),
            pltpu.VMEM((2, NB, CHUNK), jnp.int32),
            pltpu.VMEM((2, NB, CHUNK, H), _f32),
            pltpu.VMEM((RPT, H), _f32),
            pltpu.VMEM_SHARED((N_PAD, H), _f32),
            pltpu.VMEM_SHARED((N_PAD, H), _f32),
            [pltpu.SemaphoreType.DMA] * (2 * NB),
            [pltpu.SemaphoreType.DMA] * 4,
        ],
    )
    def k(hm_hbm, src_hbm, dst_hbm, zeros_hbm, out_hbm,
          src_v, dst_v, rows_v, stage_v, hm_sh, agg_sh, sems, isems):
        c = lax.axis_index("c")
        s = lax.axis_index("s")
        w = c * 16 + s
        r0 = s * RPT

        pltpu.sync_copy(zeros_hbm, stage_v)
        pltpu.sync_copy(stage_v, agg_sh.at[pl.ds(r0, RPT)])
        pltpu.sync_copy(hm_hbm.at[pl.ds(r0, RPT)], stage_v)
        pltpu.sync_copy(stage_v, hm_sh.at[pl.ds(r0, RPT)])
        plsc.subcore_barrier()

        NBATCH = CH // NB

        def load_idx(b, bb):
            return (
                pltpu.async_copy(src_hbm.at[pl.ds(w * CH + b * NB, NB)],
                                 src_v.at[bb], isems[2 * bb]),
                pltpu.async_copy(dst_hbm.at[pl.ds(w * CH + b * NB, NB)],
                                 dst_v.at[bb], isems[2 * bb + 1]),
            )

        idx_h = load_idx(0, 0)
        scat_prev = []
        for b in range(NBATCH):
            bb = b % 2
            for h in idx_h:
                h.wait()
            gathers = [
                pltpu.async_copy(hm_sh.at[src_v.at[bb, g]],
                                 rows_v.at[bb, g], sems[NB * bb + g])
                for g in range(NB)
            ]
            for h in scat_prev:
                h.wait()
            scat_prev = []
            if b + 1 < NBATCH:
                idx_h = load_idx(b + 1, (b + 1) % 2)
            for g in range(NB):
                gathers[g].wait()
                scat_prev.append(
                    pltpu.async_copy(rows_v.at[bb, g],
                                     agg_sh.at[dst_v.at[bb, g]],
                                     sems[NB * bb + g], add=True))
        for h in scat_prev:
            h.wait()
        plsc.subcore_barrier()
        pltpu.sync_copy(agg_sh.at[pl.ds(r0, RPT)],
                        out_hbm.at[pl.ds(c * N_PAD + r0, RPT)])

    return k(hm_lin, src3, dst3, zeros_tile)


def _dot(a, b):
    return jnp.dot(a, b, preferred_element_type=_f32)


def _bd(w_ref):
    w16 = w_ref[...]
    rows = jnp.concatenate([w16] * 8, axis=0)
    full = jnp.concatenate([rows] * 8, axis=1)
    ri = lax.broadcasted_iota(jnp.int32, (128, 128), 0) // H
    ci = lax.broadcasted_iota(jnp.int32, (128, 128), 1) // H
    return full * (ri == ci).astype(_f32)


def _bias(b_ref):
    return jnp.concatenate([b_ref[...]] * 8, axis=1)


def _dot_t(a, b):
    return lax.dot_general(a, b, (((0,), (0,)), ((), ())),
                           preferred_element_type=_f32)


def _pack(h):
    return jnp.concatenate([h[NR * j:NR * (j + 1), :] for j in range(8)],
                           axis=1)


def _unpack(hp):
    return jnp.concatenate([hp[:, H * j:H * (j + 1)] for j in range(8)],
                           axis=0)


def _embed_body(x_ref, we_ref, wm_ref, hp_ref, hmp_ref):
    h = _dot(x_ref[...], we_ref[...])
    hf = jnp.concatenate(
        [h, jnp.zeros((N_PAD - N, H), _f32)], axis=0)
    hp = _pack(hf)
    hp_ref[...] = hp
    hmp_ref[...] = _dot(hp, _bd(wm_ref))


def _embed_call(x, W_embed, wm0):
    return pl.pallas_call(
        _embed_body,
        out_shape=(jax.ShapeDtypeStruct((NR, 128), _f32),
                   jax.ShapeDtypeStruct((NR, 128), _f32)),
    )(x, W_embed, wm0)


def _gru_packed(hp, aggp, wz_ref, uz_ref, bz_ref, wr_ref, ur_ref, br_ref,
                wh_ref, uh_ref, bh_ref):
    z = jax.nn.sigmoid(_dot(aggp, _bd(wz_ref)) + _dot(hp, _bd(uz_ref))
                       + _bias(bz_ref))
    r = jax.nn.sigmoid(_dot(aggp, _bd(wr_ref)) + _dot(hp, _bd(ur_ref))
                       + _bias(br_ref))
    h_tilde = jnp.tanh(_dot(aggp, _bd(wh_ref)) + _dot(r * hp, _bd(uh_ref))
                       + _bias(bh_ref))
    return (1.0 - z) * hp + z * h_tilde


def _gru_body(h_ref, p_ref, wz_ref, uz_ref, bz_ref, wr_ref, ur_ref, br_ref,
              wh_ref, uh_ref, bh_ref, wm_ref, hout_ref, hm_ref):
    p = p_ref[...]
    aggp = p[:NR] + p[NR:]
    hp = _gru_packed(h_ref[...], aggp, wz_ref, uz_ref, bz_ref,
                     wr_ref, ur_ref, br_ref, wh_ref, uh_ref, bh_ref)
    hout_ref[...] = hp
    hm_ref[...] = _dot(hp, _bd(wm_ref))


def _gru_call(h, parts, *ws):
    return pl.pallas_call(
        _gru_body,
        out_shape=(jax.ShapeDtypeStruct((NR, 128), _f32),
                   jax.ShapeDtypeStruct((NR, 128), _f32)),
    )(h, parts, *ws)


def _exchange(hp, mapb_ref, wexa_ref, wexb_ref, bex_ref):
    ids_g = lax.broadcasted_iota(jnp.int32, (G, NR), 0)
    sums = jnp.zeros((G, H), _f32)
    cnt = jnp.zeros((G, 1), _f32)
    ohs = []
    for j in range(8):
        mj = mapb_ref[pl.ds(j, 1), :]
        ohj = (ids_g == mj).astype(_f32)
        ohs.append(ohj)
        sums = sums + _dot(ohj, hp[:, H * j:H * (j + 1)])
        cnt = cnt + jnp.sum(ohj, axis=1, keepdims=True)
    mean = sums / jnp.maximum(cnt, 1.0)
    pn = jnp.concatenate([_dot_t(ohj, mean) for ohj in ohs], axis=1)
    return hp + jnp.tanh(_dot(hp, _bd(wexa_ref)) + _dot(pn, _bd(wexb_ref))
                         + _bias(bex_ref))


def _gru_ex_body(h_ref, p_ref, wz_ref, uz_ref, bz_ref, wr_ref, ur_ref,
                 br_ref, wh_ref, uh_ref, bh_ref, mapb_ref, wexa_ref,
                 wexb_ref, bex_ref, wm_ref, hout_ref, hm_ref):
    p = p_ref[...]
    aggp = p[:NR] + p[NR:]
    hp = _gru_packed(h_ref[...], aggp, wz_ref, uz_ref, bz_ref,
                     wr_ref, ur_ref, br_ref, wh_ref, uh_ref, bh_ref)
    hp = _exchange(hp, mapb_ref, wexa_ref, wexb_ref, bex_ref)
    hout_ref[...] = hp
    hm_ref[...] = _dot(hp, _bd(wm_ref))


def _gru_ex_call(h, parts, *ws):
    return pl.pallas_call(
        _gru_ex_body,
        out_shape=(jax.ShapeDtypeStruct((NR, 128), _f32),
                   jax.ShapeDtypeStruct((NR, 128), _f32)),
    )(h, parts, *ws)


def _gru_ex_last_body(h_ref, p_ref, wz_ref, uz_ref, bz_ref, wr_ref, ur_ref,
                      br_ref, wh_ref, uh_ref, bh_ref, mapb_ref, wexa_ref,
                      wexb_ref, bex_ref, hout_ref):
    p = p_ref[...]
    aggp = p[:NR] + p[NR:]
    hp = _gru_packed(h_ref[...], aggp, wz_ref, uz_ref, bz_ref,
                     wr_ref, ur_ref, br_ref, wh_ref, uh_ref, bh_ref)
    hp = _exchange(hp, mapb_ref, wexa_ref, wexb_ref, bex_ref)
    hout_ref[...] = _unpack(hp)


def _gru_ex_last_call(h, parts, *ws):
    return pl.pallas_call(
        _gru_ex_last_body,
        out_shape=jax.ShapeDtypeStruct((N_PAD, H), _f32),
    )(h, parts, *ws)


def kernel(x, edge_index, node_to_graph_map, W_embed, W_msg,
           Wz, Uz, bz, Wr, Ur, br, Wh, Uh, bh, W_ex, b_ex):
    src = edge_index[0]
    dst = jnp.pad(edge_index[1], (0, E_PAD - E), constant_values=N_DUMMY)
    src = jnp.pad(src, (0, E_PAD - E))
    srcq = ((src % NR) * 8 + src // NR).reshape(NW * CH, CHUNK)
    dstq = ((dst % NR) * 8 + dst // NR).reshape(NW * CH, CHUNK)
    mapb = jnp.pad(node_to_graph_map, (0, N_PAD - N),
                   constant_values=G).reshape(8, NR)
    zeros_tile = jnp.zeros((RPT, H), _f32)

    hp, hmp = _embed_call(x, W_embed, W_msg[0])
    for l in range(L):
        parts = _sc_agg(jnp.reshape(hmp, (N_PAD, H)), srcq, dstq, zeros_tile)
        partsp = jnp.reshape(parts, (2 * NR, 128))
        gw = (Wz[l], Uz[l], bz[l].reshape(1, H), Wr[l], Ur[l],
              br[l].reshape(1, H), Wh[l], Uh[l], bh[l].reshape(1, H))
        ex_i = l // 2
        if l == L - 1:
            h_final = _gru_ex_last_call(hp, partsp, *gw, mapb,
                                        W_ex[ex_i, :H], W_ex[ex_i, H:],
                                        b_ex[ex_i].reshape(1, H))
        elif l % 2 == 1:
            hp, hmp = _gru_ex_call(hp, partsp, *gw, mapb,
                                   W_ex[ex_i, :H], W_ex[ex_i, H:],
                                   b_ex[ex_i].reshape(1, H), W_msg[l + 1])
        else:
            hp, hmp = _gru_call(hp, partsp, *gw, W_msg[l + 1])
    return h_final[:N]


# speedup vs baseline: 2.7097x; 1.0932x over previous
import functools

import jax
import jax.numpy as jnp
from jax import lax
from jax.experimental import pallas as pl
from jax.experimental.pallas import tpu as pltpu
from jax.experimental.pallas import tpu_sc as plsc

N = 10000
E = 320000
D_IN = 128
H = 16
L = 4
G = 64

NW = 32
CHUNK = 512
NB = 4
CH = 20
E_PAD = NW * CH * CHUNK
N_PAD = 10112
RPT = N_PAD // 16
NR = N_PAD // 8
N_DUMMY = N

_f32 = jnp.float32


def _sc_agg(hm_lin, src3, dst3, zeros_tile):
    mesh = plsc.VectorSubcoreMesh(core_axis_name="c", subcore_axis_name="s")

    @functools.partial(
        pl.kernel,
        mesh=mesh,
        compiler_params=pltpu.CompilerParams(use_tc_tiling_on_sc=False),
        out_type=jax.ShapeDtypeStruct((2 * N_PAD, H), _f32),
        scratch_types=[
            pltpu.VMEM((2, NB, CHUNK), jnp.int32),
            pltpu.VMEM((2, NB, CHUNK), jnp.int32),
            pltpu.VMEM((2, NB, CHUNK, H), _f32),
            pltpu.VMEM((RPT, H), _f32),
            pltpu.VMEM_SHARED((N_PAD, H), _f32),
            pltpu.VMEM_SHARED((N_PAD, H), _f32),
            [pltpu.SemaphoreType.DMA] * (2 * NB),
            [pltpu.SemaphoreType.DMA] * 4,
        ],
    )
    def k(hm_hbm, src_hbm, dst_hbm, zeros_hbm, out_hbm,
          src_v, dst_v, rows_v, stage_v, hm_sh, agg_sh, sems, isems):
        c = lax.axis_index("c")
        s = lax.axis_index("s")
        w = c * 16 + s
        r0 = s * RPT

        pltpu.sync_copy(zeros_hbm, stage_v)
        pltpu.sync_copy(stage_v, agg_sh.at[pl.ds(r0, RPT)])
        pltpu.sync_copy(hm_hbm.at[pl.ds(r0, RPT)], stage_v)
        pltpu.sync_copy(stage_v, hm_sh.at[pl.ds(r0, RPT)])
        plsc.subcore_barrier()

        NBATCH = CH // NB

        def load_idx(b, bb):
            return (
                pltpu.async_copy(src_hbm.at[pl.ds(w * CH + b * NB, NB)],
                                 src_v.at[bb], isems[2 * bb]),
                pltpu.async_copy(dst_hbm.at[pl.ds(w * CH + b * NB, NB)],
                                 dst_v.at[bb], isems[2 * bb + 1]),
            )

        idx_h = load_idx(0, 0)
        scat_prev = []
        for b in range(NBATCH):
            bb = b % 2
            for h in idx_h:
                h.wait()
            gathers = [
                pltpu.async_copy(hm_sh.at[src_v.at[bb, g]],
                                 rows_v.at[bb, g], sems[NB * bb + g])
                for g in range(NB)
            ]
            for h in scat_prev:
                h.wait()
            scat_prev = []
            if b + 1 < NBATCH:
                idx_h = load_idx(b + 1, (b + 1) % 2)
            for g in range(NB):
                gathers[g].wait()
                scat_prev.append(
                    pltpu.async_copy(rows_v.at[bb, g],
                                     agg_sh.at[dst_v.at[bb, g]],
                                     sems[NB * bb + g], add=True))
        for h in scat_prev:
            h.wait()
        plsc.subcore_barrier()
        pltpu.sync_copy(agg_sh.at[pl.ds(r0, RPT)],
                        out_hbm.at[pl.ds(c * N_PAD + r0, RPT)])

    return k(hm_lin, src3, dst3, zeros_tile)


def _dot(a, b):
    return jnp.dot(a, b, preferred_element_type=_f32)


def _bd(w_ref):
    w16 = w_ref[...]
    rows = jnp.concatenate([w16] * 8, axis=0)
    full = jnp.concatenate([rows] * 8, axis=1)
    ri = lax.broadcasted_iota(jnp.int32, (128, 128), 0) // H
    ci = lax.broadcasted_iota(jnp.int32, (128, 128), 1) // H
    return full * (ri == ci).astype(_f32)


def _bias(b_ref):
    return jnp.concatenate([b_ref[...]] * 8, axis=1)


def _dot_t(a, b):
    return lax.dot_general(a, b, (((0,), (0,)), ((), ())),
                           preferred_element_type=_f32)


def _pack(h):
    return jnp.concatenate([h[NR * j:NR * (j + 1), :] for j in range(8)],
                           axis=1)


def _unpack(hp):
    return jnp.concatenate([hp[:, H * j:H * (j + 1)] for j in range(8)],
                           axis=0)


def _embed_body(x_ref, we_ref, wm_ref, hp_ref, hmp_ref):
    h = _dot(x_ref[...], we_ref[...])
    hf = jnp.concatenate(
        [h, jnp.zeros((N_PAD - N, H), _f32)], axis=0)
    hp = _pack(hf)
    hp_ref[...] = hp
    hmp_ref[...] = _dot(hp, _bd(wm_ref))


def _embed_call(x, W_embed, wm0):
    return pl.pallas_call(
        _embed_body,
        out_shape=(jax.ShapeDtypeStruct((NR, 128), _f32),
                   jax.ShapeDtypeStruct((NR, 128), _f32)),
    )(x, W_embed, wm0)


def _gru_packed(hp, aggp, wz_ref, uz_ref, bz_ref, wr_ref, ur_ref, br_ref,
                wh_ref, uh_ref, bh_ref):
    z = jax.nn.sigmoid(_dot(aggp, _bd(wz_ref)) + _dot(hp, _bd(uz_ref))
                       + _bias(bz_ref))
    r = jax.nn.sigmoid(_dot(aggp, _bd(wr_ref)) + _dot(hp, _bd(ur_ref))
                       + _bias(br_ref))
    h_tilde = jnp.tanh(_dot(aggp, _bd(wh_ref)) + _dot(r * hp, _bd(uh_ref))
                       + _bias(bh_ref))
    return (1.0 - z) * hp + z * h_tilde


def _gru_body(h_ref, p_ref, wz_ref, uz_ref, bz_ref, wr_ref, ur_ref, br_ref,
              wh_ref, uh_ref, bh_ref, wm_ref, hout_ref, hm_ref):
    p = p_ref[...]
    aggp = p[:NR] + p[NR:]
    hp = _gru_packed(h_ref[...], aggp, wz_ref, uz_ref, bz_ref,
                     wr_ref, ur_ref, br_ref, wh_ref, uh_ref, bh_ref)
    hout_ref[...] = hp
    hm_ref[...] = _dot(hp, _bd(wm_ref))


def _gru_call(h, parts, *ws):
    return pl.pallas_call(
        _gru_body,
        out_shape=(jax.ShapeDtypeStruct((NR, 128), _f32),
                   jax.ShapeDtypeStruct((NR, 128), _f32)),
    )(h, parts, *ws)


def _exchange(hp, mapb_ref, wexa_ref, wexb_ref, bex_ref):
    ids_g = lax.broadcasted_iota(jnp.int32, (G, NR), 0)
    sums = jnp.zeros((G, H), _f32)
    cnt = jnp.zeros((G, 1), _f32)
    ohs = []
    for j in range(8):
        mj = mapb_ref[pl.ds(j, 1), :]
        ohj = (ids_g == mj).astype(_f32)
        ohs.append(ohj)
        sums = sums + _dot(ohj, hp[:, H * j:H * (j + 1)])
        cnt = cnt + jnp.sum(ohj, axis=1, keepdims=True)
    mean = sums / jnp.maximum(cnt, 1.0)
    pn = jnp.concatenate([_dot_t(ohj, mean) for ohj in ohs], axis=1)
    return hp + jnp.tanh(_dot(hp, _bd(wexa_ref)) + _dot(pn, _bd(wexb_ref))
                         + _bias(bex_ref))


def _gru_ex_body(h_ref, p_ref, wz_ref, uz_ref, bz_ref, wr_ref, ur_ref,
                 br_ref, wh_ref, uh_ref, bh_ref, mapb_ref, wexa_ref,
                 wexb_ref, bex_ref, wm_ref, hout_ref, hm_ref):
    p = p_ref[...]
    aggp = p[:NR] + p[NR:]
    hp = _gru_packed(h_ref[...], aggp, wz_ref, uz_ref, bz_ref,
                     wr_ref, ur_ref, br_ref, wh_ref, uh_ref, bh_ref)
    hp = _exchange(hp, mapb_ref, wexa_ref, wexb_ref, bex_ref)
    hout_ref[...] = hp
    hm_ref[...] = _dot(hp, _bd(wm_ref))


def _gru_ex_call(h, parts, *ws):
    return pl.pallas_call(
        _gru_ex_body,
        out_shape=(jax.ShapeDtypeStruct((NR, 128), _f32),
                   jax.ShapeDtypeStruct((NR, 128), _f32)),
    )(h, parts, *ws)


def _gru_ex_last_body(h_ref, p_ref, wz_ref, uz_ref, bz_ref, wr_ref, ur_ref,
                      br_ref, wh_ref, uh_ref, bh_ref, mapb_ref, wexa_ref,
                      wexb_ref, bex_ref, hout_ref):
    p = p_ref[...]
    aggp = p[:NR] + p[NR:]
    hp = _gru_packed(h_ref[...], aggp, wz_ref, uz_ref, bz_ref,
                     wr_ref, ur_ref, br_ref, wh_ref, uh_ref, bh_ref)
    hp = _exchange(hp, mapb_ref, wexa_ref, wexb_ref, bex_ref)
    hout_ref[...] = _unpack(hp)


def _gru_ex_last_call(h, parts, *ws):
    return pl.pallas_call(
        _gru_ex_last_body,
        out_shape=jax.ShapeDtypeStruct((N_PAD, H), _f32),
    )(h, parts, *ws)


def kernel(x, edge_index, node_to_graph_map, W_embed, W_msg,
           Wz, Uz, bz, Wr, Ur, br, Wh, Uh, bh, W_ex, b_ex):
    src = edge_index[0]
    dst = jnp.pad(edge_index[1], (0, E_PAD - E), constant_values=N_DUMMY)
    src = jnp.pad(src, (0, E_PAD - E))
    srcq = ((src % NR) * 8 + src // NR).reshape(NW * CH, CHUNK)
    dstq = ((dst % NR) * 8 + dst // NR).reshape(NW * CH, CHUNK)
    mapb = jnp.pad(node_to_graph_map, (0, N_PAD - N),
                   constant_values=G).reshape(8, NR)
    zeros_tile = jnp.zeros((RPT, H), _f32)

    hp, hmp = _embed_call(x, W_embed, W_msg[0])
    for l in range(L):
        parts = _sc_agg(jnp.reshape(hmp, (N_PAD, H)), srcq, dstq, zeros_tile)
        partsp = jnp.reshape(parts, (2 * NR, 128))
        gw = (Wz[l], Uz[l], bz[l].reshape(1, H), Wr[l], Ur[l],
              br[l].reshape(1, H), Wh[l], Uh[l], bh[l].reshape(1, H))
        ex_i = l // 2
        if l == L - 1:
            h_final = _gru_ex_last_call(hp, partsp, *gw, mapb,
                                        W_ex[ex_i, :H], W_ex[ex_i, H:],
                                        b_ex[ex_i].reshape(1, H))
        elif l % 2 == 1:
            hp, hmp = _gru_ex_call(hp, partsp, *gw, mapb,
                                   W_ex[ex_i, :H], W_ex[ex_i, H:],
                                   b_ex[ex_i].reshape(1, H), W_msg[l + 1])
        else:
            hp, hmp = _gru_call(hp, partsp, *gw, W_msg[l + 1])
    return h_final[:N]
